# baseline (speedup 1.0000x reference)
import functools

import jax
import jax.numpy as jnp
from jax import lax
from jax.experimental import pallas as pl
from jax.experimental.pallas import tpu as pltpu
from jax.experimental.pallas import tpu_sc as plsc

NC = 2
NS = 16
NW = NC * NS
LANES = 16
WIN = 80
EPS = 1e-5


def _round_up(v, m):
    return (v + m - 1) // m * m


@functools.lru_cache(maxsize=None)
def _make_norm_kernel(n_nodes, n_edges, n_rel):
    nr = n_nodes * n_rel
    per_tile_z = _round_up((nr + NS - 1) // NS, LANES)
    nr_pad = per_tile_z * NS
    ew_hist = n_edges // NS
    ew = n_edges // NW

    nwin_h = ew_hist // WIN
    nwin_c = ew // WIN
    assert nwin_h % 2 == 0 and nwin_c % 2 == 1

    mesh = plsc.VectorSubcoreMesh(core_axis_name="c", subcore_axis_name="s")

    def body(dst_hbm, et_hbm, src_hbm, norm_hbm, gidx_hbm, counts_sh, zb,
             dst0, dst1, et0, et1, src0, src1, comb0, comb1,
             gidx0, gidx1, onesb, cntb, norm0, norm1,
             seml0, seml1, sems0, sems1, semh0, semh1):
        c = lax.axis_index("c")
        s = lax.axis_index("s")
        wid = s * NC + c

        def zloop(j, _):
            zb[pl.ds(j * LANES, LANES)] = jnp.zeros((LANES,), jnp.float32)
            return 0
        lax.fori_loop(0, per_tile_z // LANES, zloop, 0)
        pltpu.sync_copy(zb, counts_sh.at[pl.ds(s * per_tile_z, per_tile_z)])

        def oloop(j, _):
            onesb[pl.ds(j * LANES, LANES)] = jnp.ones((LANES,), jnp.float32)
            return 0
        lax.fori_loop(0, WIN // LANES, oloop, 0)
        plsc.subcore_barrier()

        def h_issue(w, dstb, etb, seml):
            base = s * ew_hist + w * WIN
            pltpu.async_copy(dst_hbm.at[pl.ds(base, WIN)], dstb, seml)
            pltpu.async_copy(et_hbm.at[pl.ds(base, WIN)], etb, seml)

        def h_wait(w, dstb, etb, seml):
            base = s * ew_hist + w * WIN
            pltpu.make_async_copy(dst_hbm.at[pl.ds(base, WIN)], dstb,
                                  seml).wait()
            pltpu.make_async_copy(et_hbm.at[pl.ds(base, WIN)], etb,
                                  seml).wait()

        def h_process(w, dstb, etb, combb, semh):
            @pl.when(w >= 2)
            def _():
                pltpu.make_async_copy(onesb, counts_sh.at[combb],
                                      semh).wait()

            def cloop(j, _):
                sl = pl.ds(j * LANES, LANES)
                combb[sl] = dstb[sl] * n_rel + etb[sl]
                return 0
            lax.fori_loop(0, WIN // LANES, cloop, 0)
            pltpu.async_copy(onesb, counts_sh.at[combb], semh, add=True)

        h_issue(0, dst0, et0, seml0)

        def hloop(m, _):
            w = 2 * m
            h_wait(w, dst0, et0, seml0)
            h_issue(w + 1, dst1, et1, seml1)
            h_process(w, dst0, et0, comb0, semh0)
            h_wait(w + 1, dst1, et1, seml1)

            @pl.when(w + 2 < nwin_h)
            def _():
                h_issue(w + 2, dst0, et0, seml0)
            h_process(w + 1, dst1, et1, comb1, semh1)
            return 0
        lax.fori_loop(0, nwin_h // 2, hloop, 0)
        pltpu.make_async_copy(onesb, counts_sh.at[comb0], semh0).wait()
        pltpu.make_async_copy(onesb, counts_sh.at[comb1], semh1).wait()
        plsc.subcore_barrier()

        def c_issue(w, dstb, etb, srcb, seml):
            base = wid * ew + w * WIN
            pltpu.async_copy(dst_hbm.at[pl.ds(base, WIN)], dstb, seml)
            pltpu.async_copy(et_hbm.at[pl.ds(base, WIN)], etb, seml)
            pltpu.async_copy(src_hbm.at[pl.ds(base, WIN)], srcb, seml)

        def c_wait(w, dstb, etb, srcb, seml):
            base = wid * ew + w * WIN
            pltpu.make_async_copy(dst_hbm.at[pl.ds(base, WIN)], dstb,
                                  seml).wait()
            pltpu.make_async_copy(et_hbm.at[pl.ds(base, WIN)], etb,
                                  seml).wait()
            pltpu.make_async_copy(src_hbm.at[pl.ds(base, WIN)], srcb,
                                  seml).wait()

        def st_wait(w, normb, gidxb, sems):
            base = wid * ew + w * WIN
            pltpu.make_async_copy(normb, norm_hbm.at[pl.ds(base, WIN)],
                                  sems).wait()
            pltpu.make_async_copy(gidxb, gidx_hbm.at[pl.ds(base, WIN)],
                                  sems).wait()

        def c_process(w, dstb, etb, srcb, combb, gidxb, normb, sems):
            @pl.when(w >= 2)
            def _():
                st_wait(w - 2, normb, gidxb, sems)

            def cloop(j, _):
                sl = pl.ds(j * LANES, LANES)
                combb[sl] = dstb[sl] * n_rel + etb[sl]
                gidxb[sl] = etb[sl] * n_nodes + srcb[sl]
                return 0
            lax.fori_loop(0, WIN // LANES, cloop, 0)
            pltpu.sync_copy(counts_sh.at[combb], cntb)

            def rloop(j, _):
                sl = pl.ds(j * LANES, LANES)
                normb[sl] = 1.0 / jnp.maximum(cntb[sl], 1.0)
                return 0
            lax.fori_loop(0, WIN // LANES, rloop, 0)
            base = wid * ew + w * WIN
            pltpu.async_copy(normb, norm_hbm.at[pl.ds(base, WIN)], sems)
            pltpu.async_copy(gidxb, gidx_hbm.at[pl.ds(base, WIN)], sems)

        b0 = (dst0, et0, src0)
        b1 = (dst1, et1, src1)
        c_issue(0, *b0, seml0)

        def cwloop(m, _):
            w = 2 * m
            c_wait(w, *b0, seml0)
            c_issue(w + 1, *b1, seml1)
            c_process(w, *b0, comb0, gidx0, norm0, sems0)
            c_wait(w + 1, *b1, seml1)
            c_issue(w + 2, *b0, seml0)
            c_process(w + 1, *b1, comb1, gidx1, norm1, sems1)
            return 0
        lax.fori_loop(0, (nwin_c - 1) // 2, cwloop, 0)
        c_wait(nwin_c - 1, *b0, seml0)
        c_process(nwin_c - 1, *b0, comb0, gidx0, norm0, sems0)
        st_wait(nwin_c - 2, norm1, gidx1, sems1)
        st_wait(nwin_c - 1, norm0, gidx0, sems0)

    return pl.kernel(
        body,
        out_type=(
            jax.ShapeDtypeStruct((n_edges,), jnp.float32),
            jax.ShapeDtypeStruct((n_edges,), jnp.int32),
        ),
        mesh=mesh,
        scratch_types=[
            pltpu.VMEM_SHARED((nr_pad,), jnp.float32),
            pltpu.VMEM((per_tile_z,), jnp.float32),
        ] + [pltpu.VMEM((WIN,), jnp.int32)] * 10 + [
            pltpu.VMEM((WIN,), jnp.float32),
            pltpu.VMEM((WIN,), jnp.float32),
            pltpu.VMEM((WIN,), jnp.float32),
            pltpu.VMEM((WIN,), jnp.float32),
            pltpu.SemaphoreType.DMA,
            pltpu.SemaphoreType.DMA,
            pltpu.SemaphoreType.DMA,
            pltpu.SemaphoreType.DMA,
            pltpu.SemaphoreType.DMA,
            pltpu.SemaphoreType.DMA,
        ],
    )


@functools.lru_cache(maxsize=None)
def _make_msg_kernel(n_nodes, n_edges, d):
    ew = n_edges // NW
    nwin = ew // WIN
    n_zchunks = n_nodes // WIN
    zpasses = (n_zchunks + NS - 1) // NS

    mesh = plsc.VectorSubcoreMesh(core_axis_name="c", subcore_axis_name="s")

    def body(table_hbm, gidx_hbm, dst_hbm, norm_hbm, out_hbm,
             agg_sh, gidxv, rows0, rows1, dst0, dst1, nb0, nb1,
             semg0, semg1, semd0, semd1, semn0, semn1, semsc0, semsc1):
        c = lax.axis_index("c")
        s = lax.axis_index("s")
        wid = s * NC + c
        base = wid * ew

        def z1(e, _):
            for j in range(d // LANES):
                rows0[e, pl.ds(j * LANES, LANES)] = jnp.zeros(
                    (LANES,), jnp.float32)
            return 0
        lax.fori_loop(0, WIN, z1, 0)

        def z2(k, _):
            chunk = s + k * NS

            @pl.when(chunk < n_zchunks)
            def _():
                pltpu.sync_copy(rows0, agg_sh.at[pl.ds(chunk * WIN, WIN)])
            return 0
        lax.fori_loop(0, zpasses, z2, 0)

        pltpu.sync_copy(gidx_hbm.at[pl.ds(base, ew)], gidxv)
        plsc.subcore_barrier()

        def issue(w, rowsb, dstb, nbuf, semg, semd, semn):
            pltpu.async_copy(dst_hbm.at[pl.ds(base + w * WIN, WIN)],
                             dstb, semd)
            pltpu.async_copy(norm_hbm.at[pl.ds(base + w * WIN, WIN)],
                             nbuf, semn)
            pltpu.async_copy(table_hbm.at[gidxv.at[pl.ds(w * WIN, WIN)]],
                             rowsb, semg)

        def wait(w, rowsb, dstb, nbuf, semg, semd, semn):
            pltpu.make_async_copy(dst_hbm.at[pl.ds(base + w * WIN, WIN)],
                                  dstb, semd).wait()
            pltpu.make_async_copy(norm_hbm.at[pl.ds(base + w * WIN, WIN)],
                                  nbuf, semn).wait()
            pltpu.make_async_copy(table_hbm.at[gidxv.at[pl.ds(w * WIN, WIN)]],
                                  rowsb, semg).wait()

        def process(rowsb, dstb, nbuf, semsc):
            def sc(g, _):
                nvec = nbuf[pl.ds(g * LANES, LANES)]
                for u in range(LANES):
                    e = g * LANES + u
                    nv = jnp.broadcast_to(nvec[u], (LANES,))
                    for j in range(d // LANES):
                        sl = pl.ds(j * LANES, LANES)
                        rowsb[e, sl] = rowsb[e, sl] * nv
                return 0
            lax.fori_loop(0, WIN // LANES, sc, 0)
            pltpu.async_copy(rowsb, agg_sh.at[dstb], semsc, add=True)

        def scwait(rowsb, dstb, semsc):
            pltpu.make_async_copy(rowsb, agg_sh.at[dstb], semsc).wait()

        bufs0 = (rows0, dst0, nb0, semg0, semd0, semn0)
        bufs1 = (rows1, dst1, nb1, semg1, semd1, semn1)
        issue(0, *bufs0)

        def mloop(m, _):
            w = 2 * m
            wait(w, *bufs0)

            @pl.when(m > 0)
            def _():
                scwait(rows1, dst1, semsc1)
            issue(w + 1, *bufs1)
            process(rows0, dst0, nb0, semsc0)
            wait(w + 1, *bufs1)
            scwait(rows0, dst0, semsc0)
            issue(w + 2, *bufs0)
            process(rows1, dst1, nb1, semsc1)
            return 0
        lax.fori_loop(0, (nwin - 1) // 2, mloop, 0)
        wait(nwin - 1, *bufs0)
        scwait(rows1, dst1, semsc1)
        process(rows0, dst0, nb0, semsc0)
        scwait(rows0, dst0, semsc0)
        plsc.subcore_barrier()

        def co(k, _):
            chunk = s + k * NS

            @pl.when(chunk < n_zchunks)
            def _():
                pltpu.sync_copy(
                    agg_sh.at[pl.ds(chunk * WIN, WIN)],
                    out_hbm.at[pl.ds(c * n_nodes + chunk * WIN, WIN)])
            return 0
        lax.fori_loop(0, zpasses, co, 0)

    return pl.kernel(
        body,
        out_type=jax.ShapeDtypeStruct((2 * n_nodes, d), jnp.float32),
        mesh=mesh,
        compiler_params=pltpu.CompilerParams(
            use_tc_tiling_on_sc=(d % 128 == 0)),
        scratch_types=[
            pltpu.VMEM_SHARED((n_nodes, d), jnp.float32),
            pltpu.VMEM((ew,), jnp.int32),
            pltpu.VMEM((WIN, d), jnp.float32),
            pltpu.VMEM((WIN, d), jnp.float32),
            pltpu.VMEM((WIN,), jnp.int32),
            pltpu.VMEM((WIN,), jnp.int32),
            pltpu.VMEM((WIN,), jnp.float32),
            pltpu.VMEM((WIN,), jnp.float32),
            pltpu.SemaphoreType.DMA,
            pltpu.SemaphoreType.DMA,
            pltpu.SemaphoreType.DMA,
            pltpu.SemaphoreType.DMA,
            pltpu.SemaphoreType.DMA,
            pltpu.SemaphoreType.DMA,
            pltpu.SemaphoreType.DMA,
            pltpu.SemaphoreType.DMA,
        ],
    )


_BLK = 1000


def _mm_body(a_ref, w_ref, o_ref):
    o_ref[...] = jnp.dot(a_ref[...].astype(jnp.bfloat16), w_ref[0],
                         preferred_element_type=jnp.float32)


def _table(a, wfull):
    n, din = a.shape
    rw, _, dout = wfull.shape
    nb = n // _BLK
    return pl.pallas_call(
        _mm_body,
        grid=(rw, nb),
        in_specs=[
            pl.BlockSpec((_BLK, din), lambda r, i: (i, 0)),
            pl.BlockSpec((1, din, dout), lambda r, i: (r, 0, 0)),
        ],
        out_specs=pl.BlockSpec((_BLK, dout), lambda r, i, _nb=nb: (r * _nb + i, 0)),
        out_shape=jax.ShapeDtypeStruct((rw * n, dout), jnp.float32),
    )(a, wfull.astype(jnp.bfloat16))


def _mm_bn_body(n_rows, h_ref, st_ref, g_ref, bt_ref, w_ref, o_ref):
    st = st_ref[...]
    m = st[0:1, :] * (1.0 / n_rows)
    var = st[1:2, :] * (1.0 / n_rows) - m * m
    rinv = lax.rsqrt(var + EPS)
    a = jnp.maximum((h_ref[...] - m) * (rinv * g_ref[...]) + bt_ref[...], 0.0)
    o_ref[...] = jnp.dot(a.astype(jnp.bfloat16), w_ref[0],
                         preferred_element_type=jnp.float32)


def _table_bn(h, st, gamma, beta, wfull):
    n, din = h.shape
    rw, _, dout = wfull.shape
    nb = n // _BLK
    return pl.pallas_call(
        functools.partial(_mm_bn_body, float(n)),
        grid=(rw, nb),
        in_specs=[
            pl.BlockSpec((_BLK, din), lambda r, i: (i, 0)),
            pl.BlockSpec((2, din), lambda r, i: (0, 0)),
            pl.BlockSpec((1, din), lambda r, i: (0, 0)),
            pl.BlockSpec((1, din), lambda r, i: (0, 0)),
            pl.BlockSpec((1, din, dout), lambda r, i: (r, 0, 0)),
        ],
        out_specs=pl.BlockSpec((_BLK, dout), lambda r, i, _nb=nb: (r * _nb + i, 0)),
        out_shape=jax.ShapeDtypeStruct((rw * n, dout), jnp.float32),
    )(h, st, gamma.reshape(1, din), beta.reshape(1, din),
      wfull.astype(jnp.bfloat16))


def _comb_body(p0_ref, p1_ref, xr_ref, b_ref, h_ref, st_ref, acc_ref):
    i = pl.program_id(0)
    h = p0_ref[...] + p1_ref[...] + xr_ref[...] + b_ref[...]
    h_ref[...] = h

    @pl.when(i == 0)
    def _():
        acc_ref[...] = jnp.zeros_like(acc_ref)

    acc_ref[0:1, :] += jnp.sum(h, axis=0, keepdims=True)
    acc_ref[1:2, :] += jnp.sum(h * h, axis=0, keepdims=True)

    @pl.when(i == pl.num_programs(0) - 1)
    def _():
        st_ref[...] = acc_ref[...]


def _combine(p0, p1, xroot, b):
    n, d = p0.shape
    nb = n // _BLK
    return pl.pallas_call(
        _comb_body,
        grid=(nb,),
        in_specs=[
            pl.BlockSpec((_BLK, d), lambda i: (i, 0)),
            pl.BlockSpec((_BLK, d), lambda i: (i, 0)),
            pl.BlockSpec((_BLK, d), lambda i: (i, 0)),
            pl.BlockSpec((1, d), lambda i: (0, 0)),
        ],
        out_specs=[
            pl.BlockSpec((_BLK, d), lambda i: (i, 0)),
            pl.BlockSpec((2, d), lambda i: (0, 0)),
        ],
        out_shape=[
            jax.ShapeDtypeStruct((n, d), jnp.float32),
            jax.ShapeDtypeStruct((2, d), jnp.float32),
        ],
        scratch_shapes=[pltpu.VMEM((2, d), jnp.float32)],
    )(p0, p1, xroot, b.reshape(1, d))


def _final_body(out_d, p0_ref, p1_ref, xr_ref, b_ref, o_ref):
    s = jax.nn.sigmoid(p0_ref[...] + p1_ref[...] + xr_ref[...] + b_ref[...])
    o_ref[...] = s[:, :out_d]


def _final(p0, p1, xroot, b, out_d):
    n, d = p0.shape
    nb = n // _BLK
    return pl.pallas_call(
        functools.partial(_final_body, out_d),
        grid=(nb,),
        in_specs=[
            pl.BlockSpec((_BLK, d), lambda i: (i, 0)),
            pl.BlockSpec((_BLK, d), lambda i: (i, 0)),
            pl.BlockSpec((_BLK, d), lambda i: (i, 0)),
            pl.BlockSpec((1, d), lambda i: (0, 0)),
        ],
        out_specs=pl.BlockSpec((_BLK, out_d), lambda i: (i, 0)),
        out_shape=jax.ShapeDtypeStruct((n, out_d), jnp.float32),
    )(p0, p1, xroot, b.reshape(1, d))


def kernel(x, edge_index, edge_type, W1, root1, b1, gamma1, beta1,
           W2, root2, b2, gamma2, beta2, W3, root3, b3):
    n, _ = x.shape
    e = edge_type.shape[0]
    r = W1.shape[0]
    src = edge_index[0]
    dst = edge_index[1]
    et = edge_type

    norm, gidx = _make_norm_kernel(n, e, r)(dst, et, src)

    def conv(a_table, d):
        p = _make_msg_kernel(n, e, d)(a_table, gidx, dst, norm)
        return p[:n], p[n:]

    hid = W1.shape[2]
    out_d = W3.shape[2]

    t1 = _table(x, jnp.concatenate([W1, root1[None]], axis=0))
    p0, p1 = conv(t1, hid)
    h1, st1 = _combine(p0, p1, t1[r * n:], b1)

    t2 = _table_bn(h1, st1, gamma1, beta1,
                   jnp.concatenate([W2, root2[None]], axis=0))
    p0, p1 = conv(t2, hid)
    h2, st2 = _combine(p0, p1, t2[r * n:], b2)

    pad = ((0, 0), (0, 0), (0, hid - out_d))
    w3full = jnp.pad(jnp.concatenate([W3, root3[None]], axis=0), pad)
    t3 = _table_bn(h2, st2, gamma2, beta2, w3full)
    p0, p1 = conv(t3, hid)
    return _final(p0, p1, t3[r * n:], jnp.pad(b3, (0, hid - out_d)), out_d)


# speedup vs baseline: 1.0318x; 1.0318x over previous
import functools

import jax
import jax.numpy as jnp
from jax import lax
from jax.experimental import pallas as pl
from jax.experimental.pallas import tpu as pltpu
from jax.experimental.pallas import tpu_sc as plsc

NC = 2
NS = 16
NW = NC * NS
LANES = 16
WIN = 80
EPS = 1e-5


def _round_up(v, m):
    return (v + m - 1) // m * m


@functools.lru_cache(maxsize=None)
def _make_norm_kernel(n_nodes, n_edges, n_rel):
    nr = n_nodes * n_rel
    per_tile_z = _round_up((nr + NS - 1) // NS, LANES)
    nr_pad = per_tile_z * NS
    ew_hist = n_edges // NS
    ew = n_edges // NW

    nwin_h = ew_hist // WIN
    nwin_c = ew // WIN
    assert nwin_h % 2 == 0 and nwin_c % 2 == 1

    mesh = plsc.VectorSubcoreMesh(core_axis_name="c", subcore_axis_name="s")

    def body(dst_hbm, et_hbm, src_hbm, norm_hbm, gidx_hbm, counts_sh, zb,
             dst0, dst1, et0, et1, src0, src1, comb0, comb1,
             gidx0, gidx1, onesb, cntb, norm0, norm1,
             seml0, seml1, sems0, sems1, semh0, semh1):
        c = lax.axis_index("c")
        s = lax.axis_index("s")
        wid = s * NC + c

        def zloop(j, _):
            zb[pl.ds(j * LANES, LANES)] = jnp.zeros((LANES,), jnp.float32)
            return 0
        lax.fori_loop(0, per_tile_z // LANES, zloop, 0)
        pltpu.sync_copy(zb, counts_sh.at[pl.ds(s * per_tile_z, per_tile_z)])

        def oloop(j, _):
            onesb[pl.ds(j * LANES, LANES)] = jnp.ones((LANES,), jnp.float32)
            return 0
        lax.fori_loop(0, WIN // LANES, oloop, 0)
        plsc.subcore_barrier()

        def h_issue(w, dstb, etb, seml):
            base = s * ew_hist + w * WIN
            pltpu.async_copy(dst_hbm.at[pl.ds(base, WIN)], dstb, seml)
            pltpu.async_copy(et_hbm.at[pl.ds(base, WIN)], etb, seml)

        def h_wait(w, dstb, etb, seml):
            base = s * ew_hist + w * WIN
            pltpu.make_async_copy(dst_hbm.at[pl.ds(base, WIN)], dstb,
                                  seml).wait()
            pltpu.make_async_copy(et_hbm.at[pl.ds(base, WIN)], etb,
                                  seml).wait()

        def h_process(w, dstb, etb, combb, semh):
            @pl.when(w >= 2)
            def _():
                pltpu.make_async_copy(onesb, counts_sh.at[combb],
                                      semh).wait()

            def cloop(j, _):
                sl = pl.ds(j * LANES, LANES)
                combb[sl] = dstb[sl] * n_rel + etb[sl]
                return 0
            lax.fori_loop(0, WIN // LANES, cloop, 0)
            pltpu.async_copy(onesb, counts_sh.at[combb], semh, add=True)

        h_issue(0, dst0, et0, seml0)

        def hloop(m, _):
            w = 2 * m
            h_wait(w, dst0, et0, seml0)
            h_issue(w + 1, dst1, et1, seml1)
            h_process(w, dst0, et0, comb0, semh0)
            h_wait(w + 1, dst1, et1, seml1)

            @pl.when(w + 2 < nwin_h)
            def _():
                h_issue(w + 2, dst0, et0, seml0)
            h_process(w + 1, dst1, et1, comb1, semh1)
            return 0
        lax.fori_loop(0, nwin_h // 2, hloop, 0)
        pltpu.make_async_copy(onesb, counts_sh.at[comb0], semh0).wait()
        pltpu.make_async_copy(onesb, counts_sh.at[comb1], semh1).wait()
        plsc.subcore_barrier()

        def c_issue(w, dstb, etb, srcb, seml):
            base = wid * ew + w * WIN
            pltpu.async_copy(dst_hbm.at[pl.ds(base, WIN)], dstb, seml)
            pltpu.async_copy(et_hbm.at[pl.ds(base, WIN)], etb, seml)
            pltpu.async_copy(src_hbm.at[pl.ds(base, WIN)], srcb, seml)

        def c_wait(w, dstb, etb, srcb, seml):
            base = wid * ew + w * WIN
            pltpu.make_async_copy(dst_hbm.at[pl.ds(base, WIN)], dstb,
                                  seml).wait()
            pltpu.make_async_copy(et_hbm.at[pl.ds(base, WIN)], etb,
                                  seml).wait()
            pltpu.make_async_copy(src_hbm.at[pl.ds(base, WIN)], srcb,
                                  seml).wait()

        def st_wait(w, normb, gidxb, sems):
            base = wid * ew + w * WIN
            pltpu.make_async_copy(normb, norm_hbm.at[pl.ds(base, WIN)],
                                  sems).wait()
            pltpu.make_async_copy(gidxb, gidx_hbm.at[pl.ds(base, WIN)],
                                  sems).wait()

        def c_process(w, dstb, etb, srcb, combb, gidxb, normb, sems):
            @pl.when(w >= 2)
            def _():
                st_wait(w - 2, normb, gidxb, sems)

            def cloop(j, _):
                sl = pl.ds(j * LANES, LANES)
                combb[sl] = dstb[sl] * n_rel + etb[sl]
                gidxb[sl] = etb[sl] * n_nodes + srcb[sl]
                return 0
            lax.fori_loop(0, WIN // LANES, cloop, 0)
            pltpu.sync_copy(counts_sh.at[combb], cntb)

            def rloop(j, _):
                sl = pl.ds(j * LANES, LANES)
                normb[sl] = 1.0 / jnp.maximum(cntb[sl], 1.0)
                return 0
            lax.fori_loop(0, WIN // LANES, rloop, 0)
            base = wid * ew + w * WIN
            pltpu.async_copy(normb, norm_hbm.at[pl.ds(base, WIN)], sems)
            pltpu.async_copy(gidxb, gidx_hbm.at[pl.ds(base, WIN)], sems)

        b0 = (dst0, et0, src0)
        b1 = (dst1, et1, src1)
        c_issue(0, *b0, seml0)

        def cwloop(m, _):
            w = 2 * m
            c_wait(w, *b0, seml0)
            c_issue(w + 1, *b1, seml1)
            c_process(w, *b0, comb0, gidx0, norm0, sems0)
            c_wait(w + 1, *b1, seml1)
            c_issue(w + 2, *b0, seml0)
            c_process(w + 1, *b1, comb1, gidx1, norm1, sems1)
            return 0
        lax.fori_loop(0, (nwin_c - 1) // 2, cwloop, 0)
        c_wait(nwin_c - 1, *b0, seml0)
        c_process(nwin_c - 1, *b0, comb0, gidx0, norm0, sems0)
        st_wait(nwin_c - 2, norm1, gidx1, sems1)
        st_wait(nwin_c - 1, norm0, gidx0, sems0)

    return pl.kernel(
        body,
        out_type=(
            jax.ShapeDtypeStruct((n_edges,), jnp.float32),
            jax.ShapeDtypeStruct((n_edges,), jnp.int32),
        ),
        mesh=mesh,
        scratch_types=[
            pltpu.VMEM_SHARED((nr_pad,), jnp.float32),
            pltpu.VMEM((per_tile_z,), jnp.float32),
        ] + [pltpu.VMEM((WIN,), jnp.int32)] * 10 + [
            pltpu.VMEM((WIN,), jnp.float32),
            pltpu.VMEM((WIN,), jnp.float32),
            pltpu.VMEM((WIN,), jnp.float32),
            pltpu.VMEM((WIN,), jnp.float32),
            pltpu.SemaphoreType.DMA,
            pltpu.SemaphoreType.DMA,
            pltpu.SemaphoreType.DMA,
            pltpu.SemaphoreType.DMA,
            pltpu.SemaphoreType.DMA,
            pltpu.SemaphoreType.DMA,
        ],
    )


@functools.lru_cache(maxsize=None)
def _make_msg_kernel(n_nodes, n_edges, d):
    ew = n_edges // NW
    nwin = ew // WIN
    n_zchunks = n_nodes // WIN
    zpasses = (n_zchunks + NS - 1) // NS

    mesh = plsc.VectorSubcoreMesh(core_axis_name="c", subcore_axis_name="s")

    def body(table_hbm, gidx_hbm, dst_hbm, norm_hbm, out_hbm,
             agg_sh, gidxv, rows0, rows1, dst0, dst1, nb0, nb1,
             semg0, semg1, semd0, semd1, semn0, semn1, semsc0, semsc1):
        c = lax.axis_index("c")
        s = lax.axis_index("s")
        wid = s * NC + c
        base = wid * ew

        def z1(e, _):
            for j in range(d // LANES):
                rows0[e, pl.ds(j * LANES, LANES)] = jnp.zeros(
                    (LANES,), jnp.float32)
            return 0
        lax.fori_loop(0, WIN, z1, 0)

        def z2(k, _):
            chunk = s + k * NS

            @pl.when(chunk < n_zchunks)
            def _():
                pltpu.sync_copy(rows0, agg_sh.at[pl.ds(chunk * WIN, WIN)])
            return 0
        lax.fori_loop(0, zpasses, z2, 0)

        pltpu.sync_copy(gidx_hbm.at[pl.ds(base, ew)], gidxv)
        plsc.subcore_barrier()

        def issue(w, rowsb, dstb, nbuf, semg, semd, semn):
            pltpu.async_copy(dst_hbm.at[pl.ds(base + w * WIN, WIN)],
                             dstb, semd)
            pltpu.async_copy(norm_hbm.at[pl.ds(base + w * WIN, WIN)],
                             nbuf, semn)
            pltpu.async_copy(table_hbm.at[gidxv.at[pl.ds(w * WIN, WIN)]],
                             rowsb, semg)

        def wait(w, rowsb, dstb, nbuf, semg, semd, semn):
            pltpu.make_async_copy(dst_hbm.at[pl.ds(base + w * WIN, WIN)],
                                  dstb, semd).wait()
            pltpu.make_async_copy(norm_hbm.at[pl.ds(base + w * WIN, WIN)],
                                  nbuf, semn).wait()
            pltpu.make_async_copy(table_hbm.at[gidxv.at[pl.ds(w * WIN, WIN)]],
                                  rowsb, semg).wait()

        def process(rowsb, dstb, nbuf, semsc):
            def sc(g, _):
                nvec = nbuf[pl.ds(g * LANES, LANES)]
                for u in range(LANES):
                    e = g * LANES + u
                    nv = jnp.broadcast_to(nvec[u], (LANES,))
                    for j in range(d // LANES):
                        sl = pl.ds(j * LANES, LANES)
                        rowsb[e, sl] = rowsb[e, sl] * nv
                return 0
            lax.fori_loop(0, WIN // LANES, sc, 0)
            pltpu.async_copy(rowsb, agg_sh.at[dstb], semsc, add=True)

        def scwait(rowsb, dstb, semsc):
            pltpu.make_async_copy(rowsb, agg_sh.at[dstb], semsc).wait()

        bufs0 = (rows0, dst0, nb0, semg0, semd0, semn0)
        bufs1 = (rows1, dst1, nb1, semg1, semd1, semn1)
        issue(0, *bufs0)

        def mloop(m, _):
            w = 2 * m
            wait(w, *bufs0)

            @pl.when(m > 0)
            def _():
                scwait(rows1, dst1, semsc1)
            issue(w + 1, *bufs1)
            process(rows0, dst0, nb0, semsc0)
            wait(w + 1, *bufs1)
            scwait(rows0, dst0, semsc0)
            issue(w + 2, *bufs0)
            process(rows1, dst1, nb1, semsc1)
            return 0
        lax.fori_loop(0, (nwin - 1) // 2, mloop, 0)
        wait(nwin - 1, *bufs0)
        scwait(rows1, dst1, semsc1)
        process(rows0, dst0, nb0, semsc0)
        scwait(rows0, dst0, semsc0)
        plsc.subcore_barrier()

        def co(k, _):
            chunk = s + k * NS

            @pl.when(chunk < n_zchunks)
            def _():
                pltpu.sync_copy(
                    agg_sh.at[pl.ds(chunk * WIN, WIN)],
                    out_hbm.at[pl.ds(c * n_nodes + chunk * WIN, WIN)])
            return 0
        lax.fori_loop(0, zpasses, co, 0)

    return pl.kernel(
        body,
        out_type=jax.ShapeDtypeStruct((2 * n_nodes, d), jnp.float32),
        mesh=mesh,
        compiler_params=pltpu.CompilerParams(
            use_tc_tiling_on_sc=(d % 128 == 0)),
        scratch_types=[
            pltpu.VMEM_SHARED((n_nodes, d), jnp.float32),
            pltpu.VMEM((ew,), jnp.int32),
            pltpu.VMEM((WIN, d), jnp.float32),
            pltpu.VMEM((WIN, d), jnp.float32),
            pltpu.VMEM((WIN,), jnp.int32),
            pltpu.VMEM((WIN,), jnp.int32),
            pltpu.VMEM((WIN,), jnp.float32),
            pltpu.VMEM((WIN,), jnp.float32),
            pltpu.SemaphoreType.DMA,
            pltpu.SemaphoreType.DMA,
            pltpu.SemaphoreType.DMA,
            pltpu.SemaphoreType.DMA,
            pltpu.SemaphoreType.DMA,
            pltpu.SemaphoreType.DMA,
            pltpu.SemaphoreType.DMA,
            pltpu.SemaphoreType.DMA,
        ],
    )


_BLK = 1000


def _mm_body(a_ref, w_ref, o_ref):
    o_ref[...] = jnp.dot(a_ref[...].astype(jnp.bfloat16), w_ref[0],
                         preferred_element_type=jnp.float32)


def _table(a, wfull):
    n, din = a.shape
    rw, _, dout = wfull.shape
    nb = n // _BLK
    return pl.pallas_call(
        _mm_body,
        grid=(nb, rw),
        in_specs=[
            pl.BlockSpec((_BLK, din), lambda i, r: (i, 0)),
            pl.BlockSpec((1, din, dout), lambda i, r: (r, 0, 0)),
        ],
        out_specs=pl.BlockSpec((_BLK, dout), lambda i, r, _nb=nb: (r * _nb + i, 0)),
        out_shape=jax.ShapeDtypeStruct((rw * n, dout), jnp.float32),
    )(a, wfull.astype(jnp.bfloat16))


def _mm_bn_body(n_rows, h_ref, st_ref, g_ref, bt_ref, w_ref, o_ref):
    st = st_ref[...]
    m = st[0:1, :] * (1.0 / n_rows)
    var = st[1:2, :] * (1.0 / n_rows) - m * m
    rinv = lax.rsqrt(var + EPS)
    a = jnp.maximum((h_ref[...] - m) * (rinv * g_ref[...]) + bt_ref[...], 0.0)
    o_ref[...] = jnp.dot(a.astype(jnp.bfloat16), w_ref[0],
                         preferred_element_type=jnp.float32)


def _table_bn(h, st, gamma, beta, wfull):
    n, din = h.shape
    rw, _, dout = wfull.shape
    nb = n // _BLK
    return pl.pallas_call(
        functools.partial(_mm_bn_body, float(n)),
        grid=(nb, rw),
        in_specs=[
            pl.BlockSpec((_BLK, din), lambda i, r: (i, 0)),
            pl.BlockSpec((2, din), lambda i, r: (0, 0)),
            pl.BlockSpec((1, din), lambda i, r: (0, 0)),
            pl.BlockSpec((1, din), lambda i, r: (0, 0)),
            pl.BlockSpec((1, din, dout), lambda i, r: (r, 0, 0)),
        ],
        out_specs=pl.BlockSpec((_BLK, dout), lambda i, r, _nb=nb: (r * _nb + i, 0)),
        out_shape=jax.ShapeDtypeStruct((rw * n, dout), jnp.float32),
    )(h, st, gamma.reshape(1, din), beta.reshape(1, din),
      wfull.astype(jnp.bfloat16))


def _comb_body(p0_ref, p1_ref, xr_ref, b_ref, h_ref, st_ref, acc_ref):
    i = pl.program_id(0)
    h = p0_ref[...] + p1_ref[...] + xr_ref[...] + b_ref[...]
    h_ref[...] = h

    @pl.when(i == 0)
    def _():
        acc_ref[...] = jnp.zeros_like(acc_ref)

    acc_ref[0:1, :] += jnp.sum(h, axis=0, keepdims=True)
    acc_ref[1:2, :] += jnp.sum(h * h, axis=0, keepdims=True)

    @pl.when(i == pl.num_programs(0) - 1)
    def _():
        st_ref[...] = acc_ref[...]


def _combine(p0, p1, table, b):
    n, d = p0.shape
    nb = n // _BLK
    roff = table.shape[0] // _BLK - nb
    return pl.pallas_call(
        _comb_body,
        grid=(nb,),
        in_specs=[
            pl.BlockSpec((_BLK, d), lambda i: (i, 0)),
            pl.BlockSpec((_BLK, d), lambda i: (i, 0)),
            pl.BlockSpec((_BLK, d), lambda i, _ro=roff: (_ro + i, 0)),
            pl.BlockSpec((1, d), lambda i: (0, 0)),
        ],
        out_specs=[
            pl.BlockSpec((_BLK, d), lambda i: (i, 0)),
            pl.BlockSpec((2, d), lambda i: (0, 0)),
        ],
        out_shape=[
            jax.ShapeDtypeStruct((n, d), jnp.float32),
            jax.ShapeDtypeStruct((2, d), jnp.float32),
        ],
        scratch_shapes=[pltpu.VMEM((2, d), jnp.float32)],
    )(p0, p1, table, b.reshape(1, d))


def _final_body(out_d, p0_ref, p1_ref, xr_ref, b_ref, o_ref):
    s = jax.nn.sigmoid(p0_ref[...] + p1_ref[...] + xr_ref[...] + b_ref[...])
    o_ref[...] = s[:, :out_d]


def _final(p0, p1, table, b, out_d):
    n, d = p0.shape
    nb = n // _BLK
    roff = table.shape[0] // _BLK - nb
    return pl.pallas_call(
        functools.partial(_final_body, out_d),
        grid=(nb,),
        in_specs=[
            pl.BlockSpec((_BLK, d), lambda i: (i, 0)),
            pl.BlockSpec((_BLK, d), lambda i: (i, 0)),
            pl.BlockSpec((_BLK, d), lambda i, _ro=roff: (_ro + i, 0)),
            pl.BlockSpec((1, d), lambda i: (0, 0)),
        ],
        out_specs=pl.BlockSpec((_BLK, out_d), lambda i: (i, 0)),
        out_shape=jax.ShapeDtypeStruct((n, out_d), jnp.float32),
    )(p0, p1, table, b.reshape(1, d))


def kernel(x, edge_index, edge_type, W1, root1, b1, gamma1, beta1,
           W2, root2, b2, gamma2, beta2, W3, root3, b3):
    n, _ = x.shape
    e = edge_type.shape[0]
    r = W1.shape[0]
    src = edge_index[0]
    dst = edge_index[1]
    et = edge_type

    norm, gidx = _make_norm_kernel(n, e, r)(dst, et, src)

    def conv(a_table, d):
        p = _make_msg_kernel(n, e, d)(a_table, gidx, dst, norm)
        return p[:n], p[n:]

    hid = W1.shape[2]
    out_d = W3.shape[2]

    t1 = _table(x, jnp.concatenate([W1, root1[None]], axis=0))
    p0, p1 = conv(t1, hid)
    h1, st1 = _combine(p0, p1, t1, b1)

    t2 = _table_bn(h1, st1, gamma1, beta1,
                   jnp.concatenate([W2, root2[None]], axis=0))
    p0, p1 = conv(t2, hid)
    h2, st2 = _combine(p0, p1, t2, b2)

    pad = ((0, 0), (0, 0), (0, hid - out_d))
    w3full = jnp.pad(jnp.concatenate([W3, root3[None]], axis=0), pad)
    t3 = _table_bn(h2, st2, gamma2, beta2, w3full)
    p0, p1 = conv(t3, hid)
    return _final(p0, p1, t3, jnp.pad(b3, (0, hid - out_d)), out_d)


# speedup vs baseline: 1.1413x; 1.1061x over previous
import functools

import jax
import jax.numpy as jnp
from jax import lax
from jax.experimental import pallas as pl
from jax.experimental.pallas import tpu as pltpu
from jax.experimental.pallas import tpu_sc as plsc

NC = 2
NS = 16
NW = NC * NS
LANES = 16
WIN = 80
EPS = 1e-5


def _round_up(v, m):
    return (v + m - 1) // m * m


WINH = 400


@functools.lru_cache(maxsize=None)
def _make_norm_kernel(n_nodes, n_edges, n_rel):
    nr = n_nodes * n_rel
    per_tile_z = _round_up((nr + NS - 1) // NS, LANES)
    nr_pad = per_tile_z * NS
    ew_hist = n_edges // NS
    ew = n_edges // NW

    nwin_h = ew_hist // WINH
    nwin_c = ew // WIN
    assert nwin_h % 2 == 0 and nwin_c % 2 == 1

    mesh = plsc.VectorSubcoreMesh(core_axis_name="c", subcore_axis_name="s")

    def body(dst_hbm, et_hbm, src_hbm, norm_hbm, gidx_hbm, counts_sh, zb,
             dst0, dst1, et0, et1, src0, src1, comb0, comb1,
             gidx0, gidx1, onesb, cntb, norm0, norm1,
             seml0, seml1, sems0, sems1):
        c = lax.axis_index("c")
        s = lax.axis_index("s")
        wid = s * NC + c

        def zloop(j, _):
            zb[pl.ds(j * LANES, LANES)] = jnp.zeros((LANES,), jnp.float32)
            return 0
        lax.fori_loop(0, per_tile_z // LANES, zloop, 0)
        pltpu.sync_copy(zb, counts_sh.at[pl.ds(s * per_tile_z, per_tile_z)])

        def oloop(j, _):
            onesb[pl.ds(j * LANES, LANES)] = jnp.ones((LANES,), jnp.float32)
            return 0
        lax.fori_loop(0, WIN // LANES, oloop, 0)
        plsc.subcore_barrier()

        def h_issue(w, dstb, etb, seml):
            base = s * ew_hist + w * WINH
            pltpu.async_copy(dst_hbm.at[pl.ds(base, WINH)], dstb, seml)
            pltpu.async_copy(et_hbm.at[pl.ds(base, WINH)], etb, seml)

        def h_wait(w, dstb, etb, seml):
            base = s * ew_hist + w * WINH
            pltpu.make_async_copy(dst_hbm.at[pl.ds(base, WINH)], dstb,
                                  seml).wait()
            pltpu.make_async_copy(et_hbm.at[pl.ds(base, WINH)], etb,
                                  seml).wait()

        def h_process(dstb, etb, combb):
            def cloop(j, _):
                sl = pl.ds(j * LANES, LANES)
                combb[sl] = dstb[sl] * n_rel + etb[sl]
                return 0
            lax.fori_loop(0, WINH // LANES, cloop, 0)
            for k in range(WINH // WIN):
                pltpu.sync_copy(onesb, counts_sh.at[combb.at[pl.ds(k * WIN,
                                                                   WIN)]],
                                add=True)

        h_issue(0, dst0, et0, seml0)

        def hloop(m, _):
            w = 2 * m
            h_wait(w, dst0, et0, seml0)
            h_issue(w + 1, dst1, et1, seml1)
            h_process(dst0, et0, comb0)
            h_wait(w + 1, dst1, et1, seml1)

            @pl.when(w + 2 < nwin_h)
            def _():
                h_issue(w + 2, dst0, et0, seml0)
            h_process(dst1, et1, comb1)
            return 0
        lax.fori_loop(0, nwin_h // 2, hloop, 0)
        plsc.subcore_barrier()

        def c_issue(w, dstb, etb, srcb, seml):
            base = wid * ew + w * WIN
            pltpu.async_copy(dst_hbm.at[pl.ds(base, WIN)],
                             dstb.at[pl.ds(0, WIN)], seml)
            pltpu.async_copy(et_hbm.at[pl.ds(base, WIN)],
                             etb.at[pl.ds(0, WIN)], seml)
            pltpu.async_copy(src_hbm.at[pl.ds(base, WIN)], srcb, seml)

        def c_wait(w, dstb, etb, srcb, seml):
            base = wid * ew + w * WIN
            pltpu.make_async_copy(dst_hbm.at[pl.ds(base, WIN)],
                                  dstb.at[pl.ds(0, WIN)], seml).wait()
            pltpu.make_async_copy(et_hbm.at[pl.ds(base, WIN)],
                                  etb.at[pl.ds(0, WIN)], seml).wait()
            pltpu.make_async_copy(src_hbm.at[pl.ds(base, WIN)], srcb,
                                  seml).wait()

        def st_wait(w, normb, gidxb, sems):
            base = wid * ew + w * WIN
            pltpu.make_async_copy(normb, norm_hbm.at[pl.ds(base, WIN)],
                                  sems).wait()
            pltpu.make_async_copy(gidxb, gidx_hbm.at[pl.ds(base, WIN)],
                                  sems).wait()

        def c_process(w, dstb, etb, srcb, combb, gidxb, normb, sems):
            @pl.when(w >= 2)
            def _():
                st_wait(w - 2, normb, gidxb, sems)

            def cloop(j, _):
                sl = pl.ds(j * LANES, LANES)
                combb[sl] = dstb[sl] * n_rel + etb[sl]
                gidxb[sl] = etb[sl] * n_nodes + srcb[sl]
                return 0
            lax.fori_loop(0, WIN // LANES, cloop, 0)
            pltpu.sync_copy(counts_sh.at[combb.at[pl.ds(0, WIN)]], cntb)

            def rloop(j, _):
                sl = pl.ds(j * LANES, LANES)
                normb[sl] = 1.0 / jnp.maximum(cntb[sl], 1.0)
                return 0
            lax.fori_loop(0, WIN // LANES, rloop, 0)
            base = wid * ew + w * WIN
            pltpu.async_copy(normb, norm_hbm.at[pl.ds(base, WIN)], sems)
            pltpu.async_copy(gidxb, gidx_hbm.at[pl.ds(base, WIN)], sems)

        b0 = (dst0, et0, src0)
        b1 = (dst1, et1, src1)
        c_issue(0, *b0, seml0)

        def cwloop(m, _):
            w = 2 * m
            c_wait(w, *b0, seml0)
            c_issue(w + 1, *b1, seml1)
            c_process(w, *b0, comb0, gidx0, norm0, sems0)
            c_wait(w + 1, *b1, seml1)
            c_issue(w + 2, *b0, seml0)
            c_process(w + 1, *b1, comb1, gidx1, norm1, sems1)
            return 0
        lax.fori_loop(0, (nwin_c - 1) // 2, cwloop, 0)
        c_wait(nwin_c - 1, *b0, seml0)
        c_process(nwin_c - 1, *b0, comb0, gidx0, norm0, sems0)
        st_wait(nwin_c - 2, norm1, gidx1, sems1)
        st_wait(nwin_c - 1, norm0, gidx0, sems0)

    return pl.kernel(
        body,
        out_type=(
            jax.ShapeDtypeStruct((n_edges,), jnp.float32),
            jax.ShapeDtypeStruct((n_edges,), jnp.int32),
        ),
        mesh=mesh,
        scratch_types=[
            pltpu.VMEM_SHARED((nr_pad,), jnp.float32),
            pltpu.VMEM((per_tile_z,), jnp.float32),
            pltpu.VMEM((WINH,), jnp.int32),
            pltpu.VMEM((WINH,), jnp.int32),
            pltpu.VMEM((WINH,), jnp.int32),
            pltpu.VMEM((WINH,), jnp.int32),
            pltpu.VMEM((WIN,), jnp.int32),
            pltpu.VMEM((WIN,), jnp.int32),
            pltpu.VMEM((WINH,), jnp.int32),
            pltpu.VMEM((WINH,), jnp.int32),
            pltpu.VMEM((WIN,), jnp.int32),
            pltpu.VMEM((WIN,), jnp.int32),
            pltpu.VMEM((WIN,), jnp.float32),
            pltpu.VMEM((WIN,), jnp.float32),
            pltpu.VMEM((WIN,), jnp.float32),
            pltpu.VMEM((WIN,), jnp.float32),
            pltpu.SemaphoreType.DMA,
            pltpu.SemaphoreType.DMA,
            pltpu.SemaphoreType.DMA,
            pltpu.SemaphoreType.DMA,
        ],
    )


@functools.lru_cache(maxsize=None)
def _make_msg_kernel(n_nodes, n_edges, d):
    ew = n_edges // NW
    nwin = ew // WIN
    n_zchunks = n_nodes // WIN
    zpasses = (n_zchunks + NS - 1) // NS

    mesh = plsc.VectorSubcoreMesh(core_axis_name="c", subcore_axis_name="s")

    def body(table_hbm, gidx_hbm, dst_hbm, norm_hbm, out_hbm,
             agg_sh, gidxv, rows0, rows1, dst0, dst1, nb0, nb1,
             semg0, semg1, semd0, semd1, semn0, semn1, semsc0, semsc1):
        c = lax.axis_index("c")
        s = lax.axis_index("s")
        wid = s * NC + c
        base = wid * ew

        def z1(e, _):
            for j in range(d // LANES):
                rows0[e, pl.ds(j * LANES, LANES)] = jnp.zeros(
                    (LANES,), jnp.float32)
            return 0
        lax.fori_loop(0, WIN, z1, 0)

        def z2(k, _):
            chunk = s + k * NS

            @pl.when(chunk < n_zchunks)
            def _():
                pltpu.sync_copy(rows0, agg_sh.at[pl.ds(chunk * WIN, WIN)])
            return 0
        lax.fori_loop(0, zpasses, z2, 0)

        pltpu.sync_copy(gidx_hbm.at[pl.ds(base, ew)], gidxv)
        plsc.subcore_barrier()

        def issue(w, rowsb, dstb, nbuf, semg, semd, semn):
            pltpu.async_copy(dst_hbm.at[pl.ds(base + w * WIN, WIN)],
                             dstb, semd)
            pltpu.async_copy(norm_hbm.at[pl.ds(base + w * WIN, WIN)],
                             nbuf, semn)
            pltpu.async_copy(table_hbm.at[gidxv.at[pl.ds(w * WIN, WIN)]],
                             rowsb, semg)

        def wait(w, rowsb, dstb, nbuf, semg, semd, semn):
            pltpu.make_async_copy(dst_hbm.at[pl.ds(base + w * WIN, WIN)],
                                  dstb, semd).wait()
            pltpu.make_async_copy(norm_hbm.at[pl.ds(base + w * WIN, WIN)],
                                  nbuf, semn).wait()
            pltpu.make_async_copy(table_hbm.at[gidxv.at[pl.ds(w * WIN, WIN)]],
                                  rowsb, semg).wait()

        def process(rowsb, dstb, nbuf, semsc):
            def sc(g, _):
                nvec = nbuf[pl.ds(g * LANES, LANES)]
                for u in range(LANES):
                    e = g * LANES + u
                    nv = jnp.broadcast_to(nvec[u], (LANES,))
                    for j in range(d // LANES):
                        sl = pl.ds(j * LANES, LANES)
                        rowsb[e, sl] = rowsb[e, sl] * nv
                return 0
            lax.fori_loop(0, WIN // LANES, sc, 0)
            pltpu.async_copy(rowsb, agg_sh.at[dstb], semsc, add=True)

        def scwait(rowsb, dstb, semsc):
            pltpu.make_async_copy(rowsb, agg_sh.at[dstb], semsc).wait()

        bufs0 = (rows0, dst0, nb0, semg0, semd0, semn0)
        bufs1 = (rows1, dst1, nb1, semg1, semd1, semn1)
        issue(0, *bufs0)

        def mloop(m, _):
            w = 2 * m
            wait(w, *bufs0)

            @pl.when(m > 0)
            def _():
                scwait(rows1, dst1, semsc1)
            issue(w + 1, *bufs1)
            process(rows0, dst0, nb0, semsc0)
            wait(w + 1, *bufs1)
            scwait(rows0, dst0, semsc0)
            issue(w + 2, *bufs0)
            process(rows1, dst1, nb1, semsc1)
            return 0
        lax.fori_loop(0, (nwin - 1) // 2, mloop, 0)
        wait(nwin - 1, *bufs0)
        scwait(rows1, dst1, semsc1)
        process(rows0, dst0, nb0, semsc0)
        scwait(rows0, dst0, semsc0)
        plsc.subcore_barrier()

        def co(k, _):
            chunk = s + k * NS

            @pl.when(chunk < n_zchunks)
            def _():
                pltpu.sync_copy(
                    agg_sh.at[pl.ds(chunk * WIN, WIN)],
                    out_hbm.at[pl.ds(c * n_nodes + chunk * WIN, WIN)])
            return 0
        lax.fori_loop(0, zpasses, co, 0)

    return pl.kernel(
        body,
        out_type=jax.ShapeDtypeStruct((2 * n_nodes, d), jnp.float32),
        mesh=mesh,
        compiler_params=pltpu.CompilerParams(
            use_tc_tiling_on_sc=(d % 128 == 0)),
        scratch_types=[
            pltpu.VMEM_SHARED((n_nodes, d), jnp.float32),
            pltpu.VMEM((ew,), jnp.int32),
            pltpu.VMEM((WIN, d), jnp.float32),
            pltpu.VMEM((WIN, d), jnp.float32),
            pltpu.VMEM((WIN,), jnp.int32),
            pltpu.VMEM((WIN,), jnp.int32),
            pltpu.VMEM((WIN,), jnp.float32),
            pltpu.VMEM((WIN,), jnp.float32),
            pltpu.SemaphoreType.DMA,
            pltpu.SemaphoreType.DMA,
            pltpu.SemaphoreType.DMA,
            pltpu.SemaphoreType.DMA,
            pltpu.SemaphoreType.DMA,
            pltpu.SemaphoreType.DMA,
            pltpu.SemaphoreType.DMA,
            pltpu.SemaphoreType.DMA,
        ],
    )


_BLK = 1000


def _mm_body(a_ref, w_ref, o_ref):
    o_ref[...] = jnp.dot(a_ref[...].astype(jnp.bfloat16), w_ref[0],
                         preferred_element_type=jnp.float32)


def _table(a, wfull):
    n, din = a.shape
    rw, _, dout = wfull.shape
    nb = n // _BLK
    return pl.pallas_call(
        _mm_body,
        grid=(nb, rw),
        in_specs=[
            pl.BlockSpec((_BLK, din), lambda i, r: (i, 0)),
            pl.BlockSpec((1, din, dout), lambda i, r: (r, 0, 0)),
        ],
        out_specs=pl.BlockSpec((_BLK, dout), lambda i, r, _nb=nb: (r * _nb + i, 0)),
        out_shape=jax.ShapeDtypeStruct((rw * n, dout), jnp.float32),
    )(a, wfull.astype(jnp.bfloat16))


def _mm_bn_body(n_rows, h_ref, st_ref, g_ref, bt_ref, w_ref, o_ref):
    st = st_ref[...]
    m = st[0:1, :] * (1.0 / n_rows)
    var = st[1:2, :] * (1.0 / n_rows) - m * m
    rinv = lax.rsqrt(var + EPS)
    a = jnp.maximum((h_ref[...] - m) * (rinv * g_ref[...]) + bt_ref[...], 0.0)
    o_ref[...] = jnp.dot(a.astype(jnp.bfloat16), w_ref[0],
                         preferred_element_type=jnp.float32)


def _table_bn(h, st, gamma, beta, wfull):
    n, din = h.shape
    rw, _, dout = wfull.shape
    nb = n // _BLK
    return pl.pallas_call(
        functools.partial(_mm_bn_body, float(n)),
        grid=(nb, rw),
        in_specs=[
            pl.BlockSpec((_BLK, din), lambda i, r: (i, 0)),
            pl.BlockSpec((2, din), lambda i, r: (0, 0)),
            pl.BlockSpec((1, din), lambda i, r: (0, 0)),
            pl.BlockSpec((1, din), lambda i, r: (0, 0)),
            pl.BlockSpec((1, din, dout), lambda i, r: (r, 0, 0)),
        ],
        out_specs=pl.BlockSpec((_BLK, dout), lambda i, r, _nb=nb: (r * _nb + i, 0)),
        out_shape=jax.ShapeDtypeStruct((rw * n, dout), jnp.float32),
    )(h, st, gamma.reshape(1, din), beta.reshape(1, din),
      wfull.astype(jnp.bfloat16))


def _comb_body(p0_ref, p1_ref, xr_ref, b_ref, h_ref, st_ref, acc_ref):
    i = pl.program_id(0)
    h = p0_ref[...] + p1_ref[...] + xr_ref[...] + b_ref[...]
    h_ref[...] = h

    @pl.when(i == 0)
    def _():
        acc_ref[...] = jnp.zeros_like(acc_ref)

    acc_ref[0:1, :] += jnp.sum(h, axis=0, keepdims=True)
    acc_ref[1:2, :] += jnp.sum(h * h, axis=0, keepdims=True)

    @pl.when(i == pl.num_programs(0) - 1)
    def _():
        st_ref[...] = acc_ref[...]


def _combine(p0, p1, table, b):
    n, d = p0.shape
    nb = n // _BLK
    roff = table.shape[0] // _BLK - nb
    return pl.pallas_call(
        _comb_body,
        grid=(nb,),
        in_specs=[
            pl.BlockSpec((_BLK, d), lambda i: (i, 0)),
            pl.BlockSpec((_BLK, d), lambda i: (i, 0)),
            pl.BlockSpec((_BLK, d), lambda i, _ro=roff: (_ro + i, 0)),
            pl.BlockSpec((1, d), lambda i: (0, 0)),
        ],
        out_specs=[
            pl.BlockSpec((_BLK, d), lambda i: (i, 0)),
            pl.BlockSpec((2, d), lambda i: (0, 0)),
        ],
        out_shape=[
            jax.ShapeDtypeStruct((n, d), jnp.float32),
            jax.ShapeDtypeStruct((2, d), jnp.float32),
        ],
        scratch_shapes=[pltpu.VMEM((2, d), jnp.float32)],
    )(p0, p1, table, b.reshape(1, d))


def _final_body(out_d, p0_ref, p1_ref, xr_ref, b_ref, o_ref):
    s = jax.nn.sigmoid(p0_ref[...] + p1_ref[...] + xr_ref[...] + b_ref[...])
    o_ref[...] = s[:, :out_d]


def _final(p0, p1, table, b, out_d):
    n, d = p0.shape
    nb = n // _BLK
    roff = table.shape[0] // _BLK - nb
    return pl.pallas_call(
        functools.partial(_final_body, out_d),
        grid=(nb,),
        in_specs=[
            pl.BlockSpec((_BLK, d), lambda i: (i, 0)),
            pl.BlockSpec((_BLK, d), lambda i: (i, 0)),
            pl.BlockSpec((_BLK, d), lambda i, _ro=roff: (_ro + i, 0)),
            pl.BlockSpec((1, d), lambda i: (0, 0)),
        ],
        out_specs=pl.BlockSpec((_BLK, out_d), lambda i: (i, 0)),
        out_shape=jax.ShapeDtypeStruct((n, out_d), jnp.float32),
    )(p0, p1, table, b.reshape(1, d))


def kernel(x, edge_index, edge_type, W1, root1, b1, gamma1, beta1,
           W2, root2, b2, gamma2, beta2, W3, root3, b3):
    n, _ = x.shape
    e = edge_type.shape[0]
    r = W1.shape[0]
    src = edge_index[0]
    dst = edge_index[1]
    et = edge_type

    norm, gidx = _make_norm_kernel(n, e, r)(dst, et, src)

    def conv(a_table, d):
        p = _make_msg_kernel(n, e, d)(a_table, gidx, dst, norm)
        return p[:n], p[n:]

    hid = W1.shape[2]
    out_d = W3.shape[2]

    t1 = _table(x, jnp.concatenate([W1, root1[None]], axis=0))
    p0, p1 = conv(t1, hid)
    h1, st1 = _combine(p0, p1, t1, b1)

    t2 = _table_bn(h1, st1, gamma1, beta1,
                   jnp.concatenate([W2, root2[None]], axis=0))
    p0, p1 = conv(t2, hid)
    h2, st2 = _combine(p0, p1, t2, b2)

    pad = ((0, 0), (0, 0), (0, hid - out_d))
    w3full = jnp.pad(jnp.concatenate([W3, root3[None]], axis=0), pad)
    t3 = _table_bn(h2, st2, gamma2, beta2, w3full)
    p0, p1 = conv(t3, hid)
    return _final(p0, p1, t3, jnp.pad(b3, (0, hid - out_d)), out_d)
